# Initial kernel scaffold; baseline (speedup 1.0000x reference)
import jax
import jax.numpy as jnp
from jax.experimental import pallas as pl


def kernel(feat, segment_ids, num_segments):
    raise NotImplementedError("write your pallas kernel here")


# speedup vs baseline: 2.7376x; 2.7376x over previous
import functools

import jax
import jax.numpy as jnp
from jax import lax
from jax.experimental import pallas as pl
from jax.experimental.pallas import tpu as pltpu
from jax.experimental.pallas import tpu_sc as plsc

N = 100000
D = 128
G = 256
NC = 2
NS = 16
DHALF = D // NC
ROWS_PER_SUB = N // NS
CHUNK = 125
CHUNKS = ROWS_PER_SUB // CHUNK
G_PER_SUB = G // NS

_mesh = plsc.VectorSubcoreMesh(core_axis_name="c", subcore_axis_name="s")


@functools.partial(
    pl.kernel,
    out_type=jax.ShapeDtypeStruct((G, D), jnp.float32),
    mesh=_mesh,
    scratch_types=[
        pltpu.VMEM((CHUNKS, CHUNK), jnp.int32),
        pltpu.VMEM((CHUNK, DHALF), jnp.float32),
        pltpu.VMEM((G_PER_SUB, DHALF), jnp.float32),
        pltpu.VMEM_SHARED((G, DHALF), jnp.float32),
    ],
    compiler_params=pltpu.CompilerParams(use_tc_tiling_on_sc=False),
)
def _segsum_sc(feat_hbm, ids_hbm, out_hbm, ids_v, buf, zbuf, acc_sh):
    c = lax.axis_index("c")
    s = lax.axis_index("s")

    zeros = jnp.zeros((16,), jnp.float32)
    for r in range(G_PER_SUB):
        for d in range(DHALF // 16):
            zbuf[r, pl.ds(d * 16, 16)] = zeros
    pltpu.sync_copy(zbuf, acc_sh.at[pl.ds(s * G_PER_SUB, G_PER_SUB)])

    pltpu.sync_copy(ids_hbm.at[pl.ds(s * CHUNKS, CHUNKS)], ids_v)
    plsc.subcore_barrier()

    def body(j, carry):
        r = s * ROWS_PER_SUB + j * CHUNK
        pltpu.sync_copy(
            feat_hbm.at[pl.ds(r, CHUNK), pl.ds(c * DHALF, DHALF)], buf
        )
        pltpu.sync_copy(buf, acc_sh.at[ids_v.at[j]], add=True)
        return carry

    lax.fori_loop(0, CHUNKS, body, 0)

    plsc.subcore_barrier()
    pltpu.sync_copy(
        acc_sh.at[pl.ds(s * G_PER_SUB, G_PER_SUB)],
        out_hbm.at[pl.ds(s * G_PER_SUB, G_PER_SUB), pl.ds(c * DHALF, DHALF)],
    )


def kernel(feat, segment_ids, num_segments):
    del num_segments
    ids = segment_ids.astype(jnp.int32).reshape(NS * CHUNKS, CHUNK)
    return _segsum_sc(feat, ids)


# speedup vs baseline: 3.2680x; 1.1937x over previous
import functools

import jax
import jax.numpy as jnp
from jax import lax
from jax.experimental import pallas as pl
from jax.experimental.pallas import tpu as pltpu
from jax.experimental.pallas import tpu_sc as plsc

N = 100000
D = 128
G = 256
NC = 2
NS = 16
DHALF = D // NC
ROWS_PER_SUB = N // NS
CHUNK = 125
CHUNKS = ROWS_PER_SUB // CHUNK
G_PER_SUB = G // NS

_mesh = plsc.VectorSubcoreMesh(core_axis_name="c", subcore_axis_name="s")


@functools.partial(
    pl.kernel,
    out_type=jax.ShapeDtypeStruct((G, D), jnp.float32),
    mesh=_mesh,
    scratch_types=[
        pltpu.VMEM((CHUNKS, CHUNK), jnp.int32),
        pltpu.VMEM((CHUNK, DHALF), jnp.float32),
        pltpu.VMEM((CHUNK, DHALF), jnp.float32),
        pltpu.VMEM((G_PER_SUB, DHALF), jnp.float32),
        pltpu.VMEM_SHARED((G, DHALF), jnp.float32),
        pltpu.SemaphoreType.DMA,
        pltpu.SemaphoreType.DMA,
        pltpu.SemaphoreType.DMA,
        pltpu.SemaphoreType.DMA,
    ],
    compiler_params=pltpu.CompilerParams(use_tc_tiling_on_sc=False),
)
def _segsum_sc(
    feat_hbm, ids_hbm, out_hbm, ids_v, buf0, buf1, zbuf, acc_sh,
    gsem0, gsem1, ssem0, ssem1,
):
    c = lax.axis_index("c")
    s = lax.axis_index("s")
    base = s * ROWS_PER_SUB
    cols = pl.ds(c * DHALF, DHALF)

    def feat_at(j):
        return feat_hbm.at[pl.ds(base + j * CHUNK, CHUNK), cols]

    zeros = jnp.zeros((16,), jnp.float32)
    for r in range(G_PER_SUB):
        for d in range(DHALF // 16):
            zbuf[r, pl.ds(d * 16, 16)] = zeros
    pltpu.sync_copy(zbuf, acc_sh.at[pl.ds(s * G_PER_SUB, G_PER_SUB)])

    pltpu.sync_copy(ids_hbm.at[pl.ds(s * CHUNKS, CHUNKS)], ids_v)
    plsc.subcore_barrier()

    pltpu.async_copy(feat_at(0), buf0, gsem0)
    pltpu.async_copy(feat_at(1), buf1, gsem1)

    npair = CHUNKS // 2

    def body(i, carry):
        j0 = 2 * i
        j1 = j0 + 1
        pltpu.make_async_copy(feat_at(j0), buf0, gsem0).wait()
        sc0 = pltpu.async_copy(buf0, acc_sh.at[ids_v.at[j0]], ssem0, add=True)
        pltpu.make_async_copy(feat_at(j1), buf1, gsem1).wait()
        sc1 = pltpu.async_copy(buf1, acc_sh.at[ids_v.at[j1]], ssem1, add=True)
        sc0.wait()

        @pl.when(i + 1 < npair)
        def _():
            pltpu.async_copy(feat_at(j0 + 2), buf0, gsem0)

        sc1.wait()

        @pl.when(i + 1 < npair)
        def _():
            pltpu.async_copy(feat_at(j1 + 2), buf1, gsem1)

        return carry

    lax.fori_loop(0, npair, body, 0)

    plsc.subcore_barrier()
    pltpu.sync_copy(
        acc_sh.at[pl.ds(s * G_PER_SUB, G_PER_SUB)],
        out_hbm.at[pl.ds(s * G_PER_SUB, G_PER_SUB), pl.ds(c * DHALF, DHALF)],
    )


def kernel(feat, segment_ids, num_segments):
    del num_segments
    ids = segment_ids.astype(jnp.int32).reshape(NS * CHUNKS, CHUNK)
    return _segsum_sc(feat, ids)


# speedup vs baseline: 3.5489x; 1.0860x over previous
import functools

import jax
import jax.numpy as jnp
from jax import lax
from jax.experimental import pallas as pl
from jax.experimental.pallas import tpu as pltpu
from jax.experimental.pallas import tpu_sc as plsc

N = 100000
D = 128
G = 256
NC = 2
NS = 16
NW = NC * NS
ROWS_PER_W = N // NW
CHUNK = 125
CHUNKS_W = ROWS_PER_W // CHUNK
G_PER_SUB = G // NS

_mesh = plsc.VectorSubcoreMesh(core_axis_name="c", subcore_axis_name="s")


@functools.partial(
    pl.kernel,
    out_type=jax.ShapeDtypeStruct((NC, G, D), jnp.float32),
    mesh=_mesh,
    scratch_types=[
        pltpu.VMEM((CHUNKS_W, CHUNK), jnp.int32),
        pltpu.VMEM((CHUNK, D), jnp.float32),
        pltpu.VMEM((CHUNK, D), jnp.float32),
        pltpu.VMEM((G_PER_SUB, D), jnp.float32),
        pltpu.VMEM_SHARED((G, D), jnp.float32),
        pltpu.SemaphoreType.DMA,
        pltpu.SemaphoreType.DMA,
        pltpu.SemaphoreType.DMA,
        pltpu.SemaphoreType.DMA,
    ],
    compiler_params=pltpu.CompilerParams(use_tc_tiling_on_sc=False),
)
def _segsum_sc(
    feat_hbm, ids_hbm, out_hbm, ids_v, buf0, buf1, zbuf, acc_sh,
    gsem0, gsem1, ssem0, ssem1,
):
    c = lax.axis_index("c")
    s = lax.axis_index("s")
    w = c * NS + s
    base = w * ROWS_PER_W

    def feat_at(j):
        return feat_hbm.at[pl.ds(base + j * CHUNK, CHUNK), :]

    zeros = jnp.zeros((16,), jnp.float32)
    for r in range(G_PER_SUB):
        for d in range(D // 16):
            zbuf[r, pl.ds(d * 16, 16)] = zeros
    pltpu.sync_copy(zbuf, acc_sh.at[pl.ds(s * G_PER_SUB, G_PER_SUB)])

    pltpu.sync_copy(ids_hbm.at[pl.ds(w * CHUNKS_W, CHUNKS_W)], ids_v)
    plsc.subcore_barrier()

    pltpu.async_copy(feat_at(0), buf0, gsem0)
    pltpu.async_copy(feat_at(1), buf1, gsem1)

    npair = CHUNKS_W // 2

    def body(i, carry):
        j0 = 2 * i
        j1 = j0 + 1
        pltpu.make_async_copy(feat_at(j0), buf0, gsem0).wait()
        sc0 = pltpu.async_copy(buf0, acc_sh.at[ids_v.at[j0]], ssem0, add=True)
        pltpu.make_async_copy(feat_at(j1), buf1, gsem1).wait()
        sc1 = pltpu.async_copy(buf1, acc_sh.at[ids_v.at[j1]], ssem1, add=True)
        sc0.wait()

        @pl.when(j0 + 2 < CHUNKS_W)
        def _():
            pltpu.async_copy(feat_at(j0 + 2), buf0, gsem0)

        sc1.wait()

        @pl.when(j1 + 2 < CHUNKS_W)
        def _():
            pltpu.async_copy(feat_at(j1 + 2), buf1, gsem1)

        return carry

    lax.fori_loop(0, npair, body, 0)

    last = CHUNKS_W - 1
    pltpu.make_async_copy(feat_at(last), buf0, gsem0).wait()
    pltpu.sync_copy(buf0, acc_sh.at[ids_v.at[last]], add=True)

    plsc.subcore_barrier()
    pltpu.sync_copy(
        acc_sh.at[pl.ds(s * G_PER_SUB, G_PER_SUB)],
        out_hbm.at[c, pl.ds(s * G_PER_SUB, G_PER_SUB), :],
    )


def _combine_body(p_ref, o_ref):
    o_ref[...] = p_ref[0] + p_ref[1]


_combine = pl.pallas_call(
    _combine_body,
    out_shape=jax.ShapeDtypeStruct((G, D), jnp.float32),
)


def kernel(feat, segment_ids, num_segments):
    del num_segments
    ids = segment_ids.astype(jnp.int32).reshape(NW * CHUNKS_W, CHUNK)
    partials = _segsum_sc(feat, ids)
    return _combine(partials)
